# baseline (speedup 1.0000x reference)
import functools
import math

import jax
import jax.numpy as jnp
from jax import lax
from jax.experimental import pallas as pl
from jax.experimental.pallas import tpu as pltpu
from jax.experimental.pallas import tpu_sc as plsc

_SIZE = 32000
_PAD = 0
_SMOOTHING = 0.1
_CONF = 1.0 - _SMOOTHING
_EPS = _SMOOTHING / (_SIZE - 2)
_C = (_SIZE - 2) * _EPS * math.log(_EPS) + _CONF * math.log(_CONF)

_BR = 128

_info = plsc.get_sparse_core_info()
_NC, _NS, _L = _info.num_cores, _info.num_subcores, _info.num_lanes
_NW = _NC * _NS


def _tc_kernel(t_ref, x_ref, o_ref):
    i = pl.program_id(0)
    x = x_ref[...]
    t = t_ref[0, 0, :]
    m = (t != _PAD).astype(jnp.float32)
    rowsum = jnp.sum(x, axis=1)
    col0 = x[:, 0]
    partial = (-_EPS) * jnp.sum(rowsum * m) + _EPS * jnp.sum(col0 * m) \
        + _C * jnp.sum(m)

    @pl.when(i == 0)
    def _init():
        o_ref[...] = jnp.zeros_like(o_ref)

    o_ref[...] += jnp.full((1, 1), 1.0, jnp.float32) * partial


def _tc_part(x, target):
    n, size = x.shape
    nb = n // _BR
    t3 = target.reshape(nb, 1, _BR)
    out = pl.pallas_call(
        _tc_kernel,
        grid=(nb,),
        in_specs=[
            pl.BlockSpec((1, 1, _BR), lambda i: (i, 0, 0)),
            pl.BlockSpec((_BR, size), lambda i: (i, 0)),
        ],
        out_specs=pl.BlockSpec((1, 1), lambda i: (0, 0)),
        out_shape=jax.ShapeDtypeStruct((1, 1), jnp.float32),
    )(t3, x)
    return out[0, 0]


def _make_sc_gather(n):
    bpw = n // _NW
    mesh = plsc.VectorSubcoreMesh(core_axis_name="c", subcore_axis_name="s")

    @functools.partial(
        pl.kernel,
        mesh=mesh,
        out_type=jax.ShapeDtypeStruct((_NW, 128), jnp.float32),
        scratch_types=[
            pltpu.VMEM((bpw,), jnp.int32),
            pltpu.VMEM((bpw,), jnp.int32),
            pltpu.VMEM((bpw,), jnp.float32),
            pltpu.VMEM((128,), jnp.float32),
            pltpu.SemaphoreType.DMA,
        ],
    )
    def _sc(xflat_hbm, t_hbm, out_hbm, t_v, idx_v, val_v, acc_v, sem):
        wid = lax.axis_index("s") * _NC + lax.axis_index("c")
        base = wid * bpw
        pltpu.sync_copy(t_hbm.at[pl.ds(base, bpw)], t_v)
        for j in range(bpw // _L):
            t16 = t_v[pl.ds(j * _L, _L)]
            rows = (base + j * _L) + lax.iota(jnp.int32, _L)
            idx_v[pl.ds(j * _L, _L)] = rows * _SIZE + t16
        pltpu.async_copy(xflat_hbm.at[idx_v], val_v, sem).wait()
        acc = jnp.zeros((_L,), jnp.float32)
        for j in range(bpw // _L):
            t16 = t_v[pl.ds(j * _L, _L)]
            v16 = val_v[pl.ds(j * _L, _L)]
            acc = acc + jnp.where(t16 != _PAD, v16, 0.0)
        acc_v[pl.ds(0, _L)] = acc
        for j in range(1, 128 // _L):
            acc_v[pl.ds(j * _L, _L)] = jnp.zeros((_L,), jnp.float32)
        pltpu.sync_copy(acc_v, out_hbm.at[wid])

    return _sc


def kernel(x, target):
    n, size = x.shape
    sc_gather = _make_sc_gather(n)
    t_part = sc_gather(x.reshape(-1), target)
    a_part = _tc_part(x, target)
    return a_part + (_EPS - _CONF) * jnp.sum(t_part)


# speedup vs baseline: 3.0333x; 3.0333x over previous
import functools
import math

import jax
import jax.numpy as jnp
from jax import lax
from jax.experimental import pallas as pl
from jax.experimental.pallas import tpu as pltpu
from jax.experimental.pallas import tpu_sc as plsc

_SIZE = 32000
_PAD = 0
_SMOOTHING = 0.1
_CONF = 1.0 - _SMOOTHING
_EPS = _SMOOTHING / (_SIZE - 2)
_C = (_SIZE - 2) * _EPS * math.log(_EPS) + _CONF * math.log(_CONF)

_BR = 128

_info = plsc.get_sparse_core_info()
_NC, _NS, _L = _info.num_cores, _info.num_subcores, _info.num_lanes
_NW = _NC * _NS


def _tc_kernel(t_ref, x_ref, o_ref):
    i = pl.program_id(0)
    x = x_ref[...]
    t = t_ref[0, 0, :]
    m = (t != _PAD).astype(jnp.float32)
    ones = jnp.ones((x.shape[1], 1), jnp.float32)
    dn = (((1,), (0,)), ((), ()))
    v1 = jax.lax.dot_general(x, ones, dn, preferred_element_type=jnp.float32)
    cols = jax.lax.broadcasted_iota(jnp.int32, x.shape, 1)
    tsel = jnp.where(cols == t[:, None], x, 0.0)
    v2 = jax.lax.dot_general(tsel, ones, dn, preferred_element_type=jnp.float32)
    col0 = x[:, 0]
    partial = (-_EPS) * jnp.sum(v1[:, 0] * m) + _EPS * jnp.sum(col0 * m) \
        + (_EPS - _CONF) * jnp.sum(v2[:, 0] * m) + _C * jnp.sum(m)

    @pl.when(i == 0)
    def _init():
        o_ref[...] = jnp.zeros_like(o_ref)

    o_ref[...] += jnp.full((1, 1), 1.0, jnp.float32) * partial


def _tc_part(x, target):
    n, size = x.shape
    nb = n // _BR
    t3 = target.reshape(nb, 1, _BR)
    out = pl.pallas_call(
        _tc_kernel,
        grid=(nb,),
        in_specs=[
            pl.BlockSpec((1, 1, _BR), lambda i: (i, 0, 0)),
            pl.BlockSpec((_BR, size), lambda i: (i, 0)),
        ],
        out_specs=pl.BlockSpec((1, 1), lambda i: (0, 0)),
        out_shape=jax.ShapeDtypeStruct((1, 1), jnp.float32),
    )(t3, x)
    return out[0, 0]


def _make_sc_gather(n):
    bpw = n // _NW
    mesh = plsc.VectorSubcoreMesh(core_axis_name="c", subcore_axis_name="s")

    @functools.partial(
        pl.kernel,
        mesh=mesh,
        out_type=jax.ShapeDtypeStruct((_NW, 128), jnp.float32),
        scratch_types=[
            pltpu.VMEM((bpw,), jnp.int32),
            pltpu.VMEM((bpw,), jnp.int32),
            pltpu.VMEM((bpw,), jnp.float32),
            pltpu.VMEM((128,), jnp.float32),
            pltpu.SemaphoreType.DMA,
        ],
    )
    def _sc(xflat_hbm, t_hbm, out_hbm, t_v, idx_v, val_v, acc_v, sem):
        wid = lax.axis_index("s") * _NC + lax.axis_index("c")
        base = wid * bpw
        pltpu.sync_copy(t_hbm.at[pl.ds(base, bpw)], t_v)
        for j in range(bpw // _L):
            t16 = t_v[pl.ds(j * _L, _L)]
            rows = (base + j * _L) + lax.iota(jnp.int32, _L)
            idx_v[pl.ds(j * _L, _L)] = rows * _SIZE + t16
        pltpu.async_copy(xflat_hbm.at[idx_v], val_v, sem).wait()
        acc = jnp.zeros((_L,), jnp.float32)
        for j in range(bpw // _L):
            t16 = t_v[pl.ds(j * _L, _L)]
            v16 = val_v[pl.ds(j * _L, _L)]
            acc = acc + jnp.where(t16 != _PAD, v16, 0.0)
        acc_v[pl.ds(0, _L)] = acc
        for j in range(1, 128 // _L):
            acc_v[pl.ds(j * _L, _L)] = jnp.zeros((_L,), jnp.float32)
        pltpu.sync_copy(acc_v, out_hbm.at[wid])

    return _sc


def kernel(x, target):
    return _tc_part(x, target)
